# baseline (speedup 1.0000x reference)
import jax
import jax.numpy as jnp
from jax.experimental import pallas as pl


def _score_tc(support, b):
    def body(s_ref, b_ref, o_ref):
        o_ref[...] = jnp.tanh(s_ref[...] + b_ref[0, 0])

    return pl.pallas_call(
        body,
        out_shape=jax.ShapeDtypeStruct(support.shape, jnp.float32),
    )(support, b.reshape(1, 1))


def kernel(x, edge_index, edge_vals, W0, b):
    pre_sup = jnp.dot(x, W0)
    src = edge_index[0]
    dst = edge_index[1]
    msgs = edge_vals[:, None] * jnp.take(pre_sup, dst, axis=0)
    support = jnp.zeros((x.shape[0], 1), dtype=x.dtype).at[src].add(msgs)
    score = _score_tc(support, b)
    values, idx = jax.lax.top_k(jnp.transpose(score), 2000)
    values = jnp.transpose(values)
    new_x = jnp.squeeze(jnp.take(x, idx, axis=0), axis=0)
    new_x = new_x * values
    return new_x


# speedup vs baseline: 1.0057x; 1.0057x over previous
import functools

import jax
import jax.numpy as jnp
from jax import lax
from jax.experimental import pallas as pl
from jax.experimental.pallas import tpu as pltpu
from jax.experimental.pallas import tpu_sc as plsc

N = 10000
E = 160000
D = 256
K = 2000
NPAD = 10240
CHUNK = 640
KPAD = 2048
MINI32 = -2147483648


def _matvec(x, W0):
    def body(x_ref, w_ref, o_ref):
        o_ref[...] = jnp.dot(x_ref[...], w_ref[...],
                             preferred_element_type=jnp.float32)

    return pl.pallas_call(
        body,
        grid=(10,),
        in_specs=[pl.BlockSpec((1000, D), lambda i: (i, 0)),
                  pl.BlockSpec((D, 1), lambda i: (0, 0))],
        out_specs=pl.BlockSpec((1000, 1), lambda i: (i, 0)),
        out_shape=jax.ShapeDtypeStruct((N, 1), jnp.float32),
    )(x, W0)


def _score(support, b):
    def body(s_ref, b_ref, o_ref):
        o_ref[...] = jnp.tanh(s_ref[...] + b_ref[0, 0])

    return pl.pallas_call(
        body,
        out_shape=jax.ShapeDtypeStruct((N, 1), jnp.float32),
    )(support, b.reshape(1, 1))


def _topk_sc(score_pad):
    mesh = plsc.VectorSubcoreMesh(core_axis_name="c", subcore_axis_name="s")

    @functools.partial(
        pl.kernel,
        out_type=(jax.ShapeDtypeStruct((KPAD,), jnp.float32),
                  jax.ShapeDtypeStruct((KPAD,), jnp.int32)),
        mesh=mesh,
        scratch_types=dict(
            key_a=pltpu.VMEM_SHARED((NPAD,), jnp.int32),
            idx_a=pltpu.VMEM_SHARED((NPAD,), jnp.int32),
            key_b=pltpu.VMEM_SHARED((NPAD,), jnp.int32),
            idx_b=pltpu.VMEM_SHARED((NPAD,), jnp.int32),
            gh=pltpu.VMEM_SHARED((16, 256), jnp.int32),
            sv=pltpu.VMEM((CHUNK,), jnp.float32),
            kv=pltpu.VMEM((CHUNK,), jnp.int32),
            iv=pltpu.VMEM((CHUNK,), jnp.int32),
            hist=pltpu.VMEM((256,), jnp.int32),
            ghv=pltpu.VMEM((16, 256), jnp.int32),
            pos=pltpu.VMEM((5, 128), jnp.int32),
            smh=pltpu.SMEM((256,), jnp.int32),
            smm=pltpu.SMEM((256,), jnp.int32),
            smp=pltpu.SMEM((640,), jnp.int32),
        ),
    )
    def k(score_hbm, vals_hbm, idx_hbm, key_a, idx_a, key_b, idx_b, gh,
          sv, kv, iv, hist, ghv, pos, smh, smm, smp):
        c = lax.axis_index("c")
        s = lax.axis_index("s")
        lane = lax.iota(jnp.int32, 16)
        zero16 = jnp.zeros((16,), jnp.int32)

        def build16(scalars):
            acc = zero16
            for l, val in enumerate(scalars):
                t = lane ^ l
                nz = lax.shift_right_arithmetic(t | (0 - t), 31)
                acc = acc | (jnp.full((16,), 1, jnp.int32) * val & ~nz)
            return acc

        @pl.when(c == 0)
        def _():
            pltpu.sync_copy(score_hbm.at[pl.ds(s * CHUNK, CHUNK)], sv)
            for kk in range(CHUNK // 16):
                f = sv[pl.ds(kk * 16, 16)]
                bbits = lax.bitcast_convert_type(f, jnp.int32)
                u = jnp.where(bbits < 0, ~bbits, bbits | MINI32)
                kv[pl.ds(kk * 16, 16)] = ~u
                iv[pl.ds(kk * 16, 16)] = s * CHUNK + kk * 16 + lane
            pltpu.sync_copy(kv, key_a.at[pl.ds(s * CHUNK, CHUNK)])
            pltpu.sync_copy(iv, idx_a.at[pl.ds(s * CHUNK, CHUNK)])
            plsc.subcore_barrier()

            bufs = [(key_a, idx_a, key_b, idx_b), (key_b, idx_b, key_a, idx_a)]
            for p in range(4):
                src_k, src_i, dst_k, dst_i = bufs[p % 2]
                sh = 8 * p
                pltpu.sync_copy(src_k.at[pl.ds(s * CHUNK, CHUNK)], kv)
                pltpu.sync_copy(src_i.at[pl.ds(s * CHUNK, CHUNK)], iv)

                def zero_body(j, dead):
                    smh[j] = 0
                    return dead

                lax.fori_loop(0, 256, zero_body, 0)

                def dig_body(kk, dead):
                    dd = (lax.shift_right_logical(
                        kv[pl.ds(kk * 16, 16)], sh)) & 255
                    for l in range(16):
                        smp[kk * 16 + l] = dd[l]
                    return dead

                lax.fori_loop(0, CHUNK // 16, dig_body, 0)

                def hist_body(j, dead):
                    dd = smp[j]
                    smh[dd] = smh[dd] + 1
                    return dead

                lax.fori_loop(0, CHUNK, hist_body, 0)

                def hbuild_body(kk, dead):
                    hist[pl.ds(kk * 16, 16)] = build16(
                        [smh[kk * 16 + l] for l in range(16)])
                    return dead

                lax.fori_loop(0, 16, hbuild_body, 0)
                pltpu.sync_copy(hist, gh.at[s])
                plsc.subcore_barrier()
                pltpu.sync_copy(gh, ghv)

                sv16 = jnp.full((16,), 1, jnp.int32) * s
                for kk in range(16):
                    tot = zero16
                    mine = zero16
                    for t in range(16):
                        row = ghv[t, pl.ds(kk * 16, 16)]
                        tot = tot + row
                        tvec = jnp.full((16,), t, jnp.int32)
                        mlt = lax.shift_right_arithmetic(tvec - sv16, 31)
                        mine = mine + (row & mlt)
                    for l in range(16):
                        smh[kk * 16 + l] = tot[l]
                        smm[kk * 16 + l] = mine[l]

                def scan_body(dd, run):
                    smm[dd] = run + smm[dd]
                    return run + smh[dd]

                lax.fori_loop(0, 256, scan_body, jnp.int32(0))

                def pos_body(j, dead):
                    dd = smp[j]
                    pp = smm[dd]
                    smp[j] = pp
                    smm[dd] = pp + 1
                    return dead

                lax.fori_loop(0, CHUNK, pos_body, 0)

                def pbuild_body(m, dead):
                    cc = lax.shift_right_logical(m, 3)
                    g = m & 7
                    pos[cc, pl.ds(g * 16, 16)] = build16(
                        [smp[m * 16 + l] for l in range(16)])
                    return dead

                lax.fori_loop(0, CHUNK // 16, pbuild_body, 0)
                for cc in range(CHUNK // 128):
                    pltpu.sync_copy(kv.at[pl.ds(cc * 128, 128)],
                                    dst_k.at[pos.at[cc]])
                    pltpu.sync_copy(iv.at[pl.ds(cc * 128, 128)],
                                    dst_i.at[pos.at[cc]])
                plsc.subcore_barrier()

            pltpu.sync_copy(key_a.at[pl.ds(s * 128, 128)], kv.at[pl.ds(0, 128)])
            pltpu.sync_copy(idx_a.at[pl.ds(s * 128, 128)], iv.at[pl.ds(0, 128)])
            for kk in range(8):
                u = ~kv[pl.ds(kk * 16, 16)]
                bbits = jnp.where(u < 0, u ^ MINI32, ~u)
                sv[pl.ds(kk * 16, 16)] = lax.bitcast_convert_type(
                    bbits, jnp.float32)
            pltpu.sync_copy(sv.at[pl.ds(0, 128)],
                            vals_hbm.at[pl.ds(s * 128, 128)])
            pltpu.sync_copy(iv.at[pl.ds(0, 128)],
                            idx_hbm.at[pl.ds(s * 128, 128)])

    return k(score_pad)


def _gather_scale(x, idx, vals):
    mesh = plsc.VectorSubcoreMesh(core_axis_name="c", subcore_axis_name="s")
    RPW = KPAD // 32

    @functools.partial(
        pl.kernel,
        out_type=jax.ShapeDtypeStruct((KPAD, D), jnp.float32),
        mesh=mesh,
        scratch_types=dict(
            idx_v=pltpu.VMEM((RPW,), jnp.int32),
            val_v=pltpu.VMEM((RPW,), jnp.float32),
            rows=pltpu.VMEM((RPW, D), jnp.float32),
            sem=pltpu.SemaphoreType.DMA,
        ),
    )
    def k(x_hbm, idx_hbm, vals_hbm, out_hbm, idx_v, val_v, rows, sem):
        c = lax.axis_index("c")
        s = lax.axis_index("s")
        w = s * 2 + c
        base = w * RPW
        pltpu.sync_copy(idx_hbm.at[pl.ds(base, RPW)], idx_v)
        pltpu.sync_copy(vals_hbm.at[pl.ds(base, RPW)], val_v)
        pltpu.async_copy(x_hbm.at[idx_v], rows, sem).wait()
        for g in range(RPW // 16):
            vv = val_v[pl.ds(g * 16, 16)]
            for r in range(16):
                vscale = jnp.full((16,), vv[r], jnp.float32)
                row = g * 16 + r
                for kk in range(D // 16):
                    rows[row, pl.ds(kk * 16, 16)] = (
                        rows[row, pl.ds(kk * 16, 16)] * vscale)
        pltpu.sync_copy(rows, out_hbm.at[pl.ds(base, RPW)])

    return k(x, idx, vals)


def kernel(x, edge_index, edge_vals, W0, b):
    pre_sup = _matvec(x, W0)
    src = edge_index[0]
    dst = edge_index[1]
    msgs = edge_vals[:, None] * jnp.take(pre_sup, dst, axis=0)
    support = jnp.zeros((N, 1), dtype=x.dtype).at[src].add(msgs)
    score = _score(support, b)
    score_pad = jnp.concatenate(
        [score[:, 0], jnp.full((NPAD - N,), -2.0, jnp.float32)])
    vals, idx = _topk_sc(score_pad)
    new_x = _gather_scale(x, idx, vals)
    return new_x[:K]
